# Initial kernel scaffold; baseline (speedup 1.0000x reference)
import jax
import jax.numpy as jnp
from jax.experimental import pallas as pl


def kernel(x, edge_index, edge_attr, W_u, a, W_e, W_m, W1, b1, W2, b2, W3, b3, W4, b4):
    raise NotImplementedError("write your pallas kernel here")


# speedup vs baseline: 13.5159x; 13.5159x over previous
import functools

import jax
import jax.numpy as jnp
from jax import lax
from jax.experimental import pallas as pl
from jax.experimental.pallas import tpu as pltpu
from jax.experimental.pallas import tpu_sc as plsc

N = 10000
E = 320000
H = 128
NC = 2
NS = 16
NW = NC * NS
EPW = E // NW
LANES = 16
BN = 2000
NB = N // BN
BE8 = 5000


def _nodes_body(x_ref, wu_ref, a2_ref, g_ref, p_ref):
    u = jnp.dot(x_ref[...], wu_ref[...], preferred_element_type=jnp.float32)
    hp = jnp.where(u > 0, u, 0.01 * u)
    g_ref[...] = jnp.where(u > 0, u, 0.0001 * u)
    p_ref[...] = jnp.dot(hp, a2_ref[...], preferred_element_type=jnp.float32)


def _nodes_call(x, W_u, A2):
    return pl.pallas_call(
        _nodes_body,
        grid=(NB,),
        in_specs=[
            pl.BlockSpec((BN, H), lambda i: (i, 0)),
            pl.BlockSpec((H, H), lambda i: (0, 0)),
            pl.BlockSpec((H, 2), lambda i: (0, 0)),
        ],
        out_specs=[
            pl.BlockSpec((BN, H), lambda i: (i, 0)),
            pl.BlockSpec((BN, 2), lambda i: (i, 0)),
        ],
        out_shape=[
            jax.ShapeDtypeStruct((N, H), jnp.float32),
            jax.ShapeDtypeStruct((N, 2), jnp.float32),
        ],
    )(x, W_u, A2)


def _edges_body(ea8_ref, we_ref, wm_ref, t_ref):
    wem = jnp.dot(we_ref[...], wm_ref[...], preferred_element_type=jnp.float32)
    wem8 = jnp.concatenate([wem] * 8, axis=0)
    rows = lax.broadcasted_iota(jnp.int32, (H, 8), 0)
    cols = lax.broadcasted_iota(jnp.int32, (H, 8), 1)
    K = jnp.where(rows // 16 == cols, wem8, 0.0)
    t_ref[...] = jnp.dot(ea8_ref[...], K, preferred_element_type=jnp.float32)


def _edges_call(ea8, W_e, W_m):
    d_edge = W_e.shape[0]
    er = E // 8
    return pl.pallas_call(
        _edges_body,
        grid=(er // BE8,),
        in_specs=[
            pl.BlockSpec((BE8, H), lambda i: (i, 0)),
            pl.BlockSpec((d_edge, H), lambda i: (0, 0)),
            pl.BlockSpec((H, 1), lambda i: (0, 0)),
        ],
        out_specs=pl.BlockSpec((BE8, 8), lambda i: (i, 0)),
        out_shape=jax.ShapeDtypeStruct((er, 8), jnp.float32),
    )(ea8, W_e, W_m)


def _sc_edge_body(src_hbm, dst_hbm, t_hbm, p1_hbm, p2_hbm, w_out, m_out,
                  src_v, dst_v, t_v, p1_v, p2_v, m_v, w_v, mrow_v):
    wid = lax.axis_index("s") * NC + lax.axis_index("c")
    base = wid * EPW
    pltpu.sync_copy(src_hbm.at[pl.ds(base, EPW)], src_v)
    pltpu.sync_copy(dst_hbm.at[pl.ds(base, EPW)], dst_v)
    pltpu.sync_copy(t_hbm.at[pl.ds(base, EPW)], t_v)
    pltpu.sync_copy(p1_hbm, p1_v)
    pltpu.sync_copy(p2_hbm, p2_v)

    niter = EPW // LANES

    def body1(i, mx):
        sl = pl.ds(pl.multiple_of(i * LANES, LANES), LANES)
        si = src_v[sl]
        di = dst_v[sl]
        sc = plsc.load_gather(p1_v, [si]) + plsc.load_gather(p2_v, [di])
        e = jnp.exp(-jnp.abs(sc))
        sig = jnp.where(sc >= 0, 1.0 / (1.0 + e), e / (1.0 + e))
        m = sig * t_v[sl]
        m_v[sl] = m
        return jnp.maximum(mx, m)

    mx0 = jnp.full((LANES,), -jnp.inf, dtype=jnp.float32)
    mx = lax.fori_loop(0, niter, body1, mx0)
    mt = jnp.max(mx)

    def bzero(i, carry):
        w_v[pl.ds(pl.multiple_of(i * LANES, LANES), LANES)] = (
            jnp.zeros((LANES,), jnp.float32))
        return carry

    lax.fori_loop(0, N // LANES, bzero, 0)

    def body2(i, carry):
        sl = pl.ds(pl.multiple_of(i * LANES, LANES), LANES)
        si = src_v[sl]
        ev = jnp.exp(m_v[sl] - mt)
        plsc.addupdate_scatter(w_v, [si], ev)
        return carry

    lax.fori_loop(0, niter, body2, 0)

    mrow_v[...] = jnp.full((LANES,), mt, dtype=jnp.float32)
    for b in range(NB):
        pltpu.sync_copy(w_v.at[pl.ds(b * BN, BN)], w_out.at[b, wid])
    pltpu.sync_copy(mrow_v, m_out.at[wid])


def _sc_call(src, dst, t, p1, p2):
    mesh = plsc.VectorSubcoreMesh(core_axis_name="c", subcore_axis_name="s")
    f = functools.partial(
        pl.kernel,
        mesh=mesh,
        compiler_params=pltpu.CompilerParams(
            needs_layout_passes=False, use_tc_tiling_on_sc=False),
        out_type=[
            jax.ShapeDtypeStruct((NB, NW, BN), jnp.float32),
            jax.ShapeDtypeStruct((NW, LANES), jnp.float32),
        ],
        scratch_types=[
            pltpu.VMEM((EPW,), jnp.int32),
            pltpu.VMEM((EPW,), jnp.int32),
            pltpu.VMEM((EPW,), jnp.float32),
            pltpu.VMEM((N,), jnp.float32),
            pltpu.VMEM((N,), jnp.float32),
            pltpu.VMEM((EPW,), jnp.float32),
            pltpu.VMEM((N,), jnp.float32),
            pltpu.VMEM((LANES,), jnp.float32),
        ],
    )(_sc_edge_body)
    return f(src, dst, t, p1, p2)


def _final_body(w_ref, mloc_ref, g_ref, w1, b1, w2, b2, w3, b3, w4, b4,
                out_ref, acc_ref, accs_ref):
    i = pl.program_id(0)

    @pl.when(i == 0)
    def _init():
        acc_ref[...] = jnp.zeros_like(acc_ref)
        accs_ref[0, 0] = 0.0

    mloc = mloc_ref[...]
    gmax = jnp.max(mloc)
    c = jnp.exp(mloc[:, 0:1] - gmax)
    wblk = w_ref[0]
    cw = jnp.sum(wblk * c, axis=0, keepdims=True)
    acc_ref[...] += jnp.dot(cw, g_ref[...], preferred_element_type=jnp.float32)
    accs_ref[0, 0] += jnp.sum(cw)

    @pl.when(i == pl.num_programs(0) - 1)
    def _finish():
        hg = acc_ref[...] / accs_ref[0, 0]
        o = jnp.dot(hg, w1[...], preferred_element_type=jnp.float32) + b1[...]
        o = jnp.maximum(o, 0.0)
        o = jnp.dot(o, w2[...], preferred_element_type=jnp.float32) + b2[...]
        o = jnp.maximum(o, 0.0)
        o = jnp.dot(o, w3[...], preferred_element_type=jnp.float32) + b3[...]
        o = jnp.maximum(o, 0.0)
        out_ref[...] = (jnp.dot(o, w4[...], preferred_element_type=jnp.float32)
                        + b4[...])


def _final_call(w5, Mloc, g, W1, b1, W2, b2, W3, b3, W4, b4):
    full = lambda i: (0, 0)
    return pl.pallas_call(
        _final_body,
        grid=(NB,),
        in_specs=[
            pl.BlockSpec((1, NW, BN), lambda i: (i, 0, 0)),
            pl.BlockSpec((NW, LANES), full),
            pl.BlockSpec((BN, H), lambda i: (i, 0)),
            pl.BlockSpec(W1.shape, full),
            pl.BlockSpec(b1.shape, full),
            pl.BlockSpec(W2.shape, full),
            pl.BlockSpec(b2.shape, full),
            pl.BlockSpec(W3.shape, full),
            pl.BlockSpec(b3.shape, full),
            pl.BlockSpec(W4.shape, full),
            pl.BlockSpec(b4.shape, full),
        ],
        out_specs=pl.BlockSpec((1, 1), full),
        out_shape=jax.ShapeDtypeStruct((1, 1), jnp.float32),
        scratch_shapes=[
            pltpu.VMEM((1, H), jnp.float32),
            pltpu.SMEM((1, 1), jnp.float32),
        ],
    )(w5, Mloc, g, W1, b1, W2, b2, W3, b3, W4, b4)


def kernel(x, edge_index, edge_attr, W_u, a, W_e, W_m,
           W1, b1, W2, b2, W3, b3, W4, b4):
    src = edge_index[0]
    dst = edge_index[1]
    A2 = jnp.concatenate([a[:H], a[H:]], axis=1)

    g, P = _nodes_call(x, W_u, A2)
    t = _edges_call(edge_attr.reshape(E // 8, 8 * 16), W_e, W_m)

    w5, Mloc = _sc_call(src, dst, t.reshape(E), P[:, 0], P[:, 1])

    return _final_call(w5, Mloc, g,
                       W1, b1.reshape(1, -1), W2, b2.reshape(1, -1),
                       W3, b3.reshape(1, -1), W4, b4.reshape(1, -1))


# speedup vs baseline: 14.0475x; 1.0393x over previous
import functools

import jax
import jax.numpy as jnp
from jax import lax
from jax.experimental import pallas as pl
from jax.experimental.pallas import tpu as pltpu
from jax.experimental.pallas import tpu_sc as plsc

N = 10000
E = 320000
H = 128
NC = 2
NS = 16
NW = NC * NS
EPW = E // NW
LANES = 16
BN = 2000
NB = N // BN
GAB = 10
BNA = N // GAB
BE8 = (E // 8) // GAB
UNROLL = 5
NITER = EPW // LANES


def _ab_body(x_ref, wu_ref, a2_ref, ea8_ref, we_ref, wm_ref,
             g_ref, p_ref, t_ref):
    u = jnp.dot(x_ref[...], wu_ref[...], preferred_element_type=jnp.float32)
    hp = jnp.where(u > 0, u, 0.01 * u)
    g_ref[...] = jnp.where(u > 0, u, 0.0001 * u)
    p_ref[...] = jnp.dot(hp, a2_ref[...], preferred_element_type=jnp.float32)
    wem = jnp.dot(we_ref[...], wm_ref[...], preferred_element_type=jnp.float32)
    wem8 = jnp.concatenate([wem] * 8, axis=0)
    rows = lax.broadcasted_iota(jnp.int32, (H, 8), 0)
    cols = lax.broadcasted_iota(jnp.int32, (H, 8), 1)
    km = jnp.where(rows // 16 == cols, wem8, 0.0)
    t_ref[...] = jnp.dot(ea8_ref[...], km, preferred_element_type=jnp.float32)


def _ab_call(x, W_u, A2, ea8, W_e, W_m):
    d_edge = W_e.shape[0]
    er = E // 8
    return pl.pallas_call(
        _ab_body,
        grid=(GAB,),
        in_specs=[
            pl.BlockSpec((BNA, H), lambda i: (i, 0)),
            pl.BlockSpec((H, H), lambda i: (0, 0)),
            pl.BlockSpec((H, 2), lambda i: (0, 0)),
            pl.BlockSpec((BE8, H), lambda i: (i, 0)),
            pl.BlockSpec((d_edge, H), lambda i: (0, 0)),
            pl.BlockSpec((H, 1), lambda i: (0, 0)),
        ],
        out_specs=[
            pl.BlockSpec((BNA, H), lambda i: (i, 0)),
            pl.BlockSpec((BNA, 2), lambda i: (i, 0)),
            pl.BlockSpec((BE8, 8), lambda i: (i, 0)),
        ],
        out_shape=[
            jax.ShapeDtypeStruct((N, H), jnp.float32),
            jax.ShapeDtypeStruct((N, 2), jnp.float32),
            jax.ShapeDtypeStruct((er, 8), jnp.float32),
        ],
    )(x, W_u, A2, ea8, W_e, W_m)


def _sc_edge_body(ei_hbm, t_hbm, pq_hbm, w_out, m_out,
                  src_v, dst_v, t_v, pq_v, m_v, w_v, mrow_v):
    wid = lax.axis_index("s") * NC + lax.axis_index("c")
    base = wid * EPW
    pltpu.sync_copy(ei_hbm.at[pl.ds(base, EPW)], src_v)
    pltpu.sync_copy(ei_hbm.at[pl.ds(E + base, EPW)], dst_v)
    pltpu.sync_copy(t_hbm.at[pl.ds(base, EPW)], t_v)
    pltpu.sync_copy(pq_hbm, pq_v)

    def body1(i, mx):
        for j in range(UNROLL):
            sl = pl.ds(pl.multiple_of((i * UNROLL + j) * LANES, LANES), LANES)
            si = src_v[sl]
            di = dst_v[sl]
            sc = (plsc.load_gather(pq_v, [si * 2])
                  + plsc.load_gather(pq_v, [di * 2 + 1]))
            e = jnp.exp(-jnp.abs(sc))
            sig = jnp.where(sc >= 0, 1.0 / (1.0 + e), e / (1.0 + e))
            m = sig * t_v[sl]
            m_v[sl] = m
            w_v[sl] = jnp.zeros((LANES,), jnp.float32)
            mx = jnp.maximum(mx, m)
        return mx

    mx0 = jnp.full((LANES,), -jnp.inf, dtype=jnp.float32)
    mx = lax.fori_loop(0, NITER // UNROLL, body1, mx0)
    mt = jnp.max(mx)

    def body2(i, carry):
        for j in range(UNROLL):
            sl = pl.ds(pl.multiple_of((i * UNROLL + j) * LANES, LANES), LANES)
            si = src_v[sl]
            ev = jnp.exp(m_v[sl] - mt)
            plsc.addupdate_scatter(w_v, [si], ev)
        return carry

    lax.fori_loop(0, NITER // UNROLL, body2, 0)

    mrow_v[...] = jnp.full((LANES,), mt, dtype=jnp.float32)
    for b in range(NB):
        pltpu.sync_copy(w_v.at[pl.ds(b * BN, BN)], w_out.at[b, wid])
    pltpu.sync_copy(mrow_v, m_out.at[wid])


def _sc_call(ei_flat, t, pq):
    mesh = plsc.VectorSubcoreMesh(core_axis_name="c", subcore_axis_name="s")
    f = functools.partial(
        pl.kernel,
        mesh=mesh,
        compiler_params=pltpu.CompilerParams(
            needs_layout_passes=False, use_tc_tiling_on_sc=False),
        out_type=[
            jax.ShapeDtypeStruct((NB, NW, BN), jnp.float32),
            jax.ShapeDtypeStruct((NW, LANES), jnp.float32),
        ],
        scratch_types=[
            pltpu.VMEM((EPW,), jnp.int32),
            pltpu.VMEM((EPW,), jnp.int32),
            pltpu.VMEM((EPW,), jnp.float32),
            pltpu.VMEM((2 * N,), jnp.float32),
            pltpu.VMEM((EPW,), jnp.float32),
            pltpu.VMEM((N,), jnp.float32),
            pltpu.VMEM((LANES,), jnp.float32),
        ],
    )(_sc_edge_body)
    return f(ei_flat, t, pq)


def _final_body(w_ref, mloc_ref, g_ref, w1, b1, w2, b2, w3, b3, w4, b4,
                out_ref, acc_ref, accs_ref):
    i = pl.program_id(0)

    @pl.when(i == 0)
    def _init():
        acc_ref[...] = jnp.zeros_like(acc_ref)
        accs_ref[0, 0] = 0.0

    mloc = mloc_ref[...]
    gmax = jnp.max(mloc)
    c = jnp.exp(mloc[:, 0:1] - gmax)
    wblk = w_ref[0]
    cw = jnp.sum(wblk * c, axis=0, keepdims=True)
    acc_ref[...] += jnp.dot(cw, g_ref[...], preferred_element_type=jnp.float32)
    accs_ref[0, 0] += jnp.sum(cw)

    @pl.when(i == pl.num_programs(0) - 1)
    def _finish():
        hg = acc_ref[...] / accs_ref[0, 0]
        o = jnp.dot(hg, w1[...], preferred_element_type=jnp.float32) + b1[...]
        o = jnp.maximum(o, 0.0)
        o = jnp.dot(o, w2[...], preferred_element_type=jnp.float32) + b2[...]
        o = jnp.maximum(o, 0.0)
        o = jnp.dot(o, w3[...], preferred_element_type=jnp.float32) + b3[...]
        o = jnp.maximum(o, 0.0)
        out_ref[...] = (jnp.dot(o, w4[...], preferred_element_type=jnp.float32)
                        + b4[...])


def _final_call(w5, Mloc, g, W1, b1, W2, b2, W3, b3, W4, b4):
    full = lambda i: (0, 0)
    return pl.pallas_call(
        _final_body,
        grid=(NB,),
        in_specs=[
            pl.BlockSpec((1, NW, BN), lambda i: (i, 0, 0)),
            pl.BlockSpec((NW, LANES), full),
            pl.BlockSpec((BN, H), lambda i: (i, 0)),
            pl.BlockSpec(W1.shape, full),
            pl.BlockSpec(b1.shape, full),
            pl.BlockSpec(W2.shape, full),
            pl.BlockSpec(b2.shape, full),
            pl.BlockSpec(W3.shape, full),
            pl.BlockSpec(b3.shape, full),
            pl.BlockSpec(W4.shape, full),
            pl.BlockSpec(b4.shape, full),
        ],
        out_specs=pl.BlockSpec((1, 1), full),
        out_shape=jax.ShapeDtypeStruct((1, 1), jnp.float32),
        scratch_shapes=[
            pltpu.VMEM((1, H), jnp.float32),
            pltpu.SMEM((1, 1), jnp.float32),
        ],
    )(w5, Mloc, g, W1, b1, W2, b2, W3, b3, W4, b4)


def kernel(x, edge_index, edge_attr, W_u, a, W_e, W_m,
           W1, b1, W2, b2, W3, b3, W4, b4):
    A2 = jnp.concatenate([a[:H], a[H:]], axis=1)

    g, P, t8 = _ab_call(x, W_u, A2, edge_attr.reshape(E // 8, 8 * 16),
                        W_e, W_m)

    w5, Mloc = _sc_call(edge_index.reshape(2 * E), t8.reshape(E),
                        P.reshape(2 * N))

    return _final_call(w5, Mloc, g,
                       W1, b1.reshape(1, -1), W2, b2.reshape(1, -1),
                       W3, b3.reshape(1, -1), W4, b4.reshape(1, -1))
